# Initial kernel scaffold; baseline (speedup 1.0000x reference)
import jax
import jax.numpy as jnp
from jax.experimental import pallas as pl


def kernel(x, params, edge_index, batch, target):
    raise NotImplementedError("write your pallas kernel here")


# speedup vs baseline: 6.5975x; 6.5975x over previous
import functools

import jax
import jax.numpy as jnp
from jax import lax
from jax.experimental import pallas as pl
from jax.experimental.pallas import tpu as pltpu
from jax.experimental.pallas import tpu_sc as plsc

N = 50000
E = 800000
B = 128
NB = 16
BLK = 3136
N_PAD = NB * BLK
NW = 32
ECH = 128
NCHUNK = 196
EPW = NCHUNK * ECH
E_PAD = NW * EPW
EPS = 1e-5
HI = lax.Precision.HIGHEST


def _sc_body(y_hbm, src_hbm, dst_hbm, out_hbm,
             s_idx, d_idx, rows0, rows1, out_acc, sem0, sem1):
    cid = lax.axis_index("c")
    sid = lax.axis_index("s")
    wid = cid * 16 + sid
    base = sid * BLK

    z16 = jnp.zeros((16,), jnp.float32)

    def zf(i, carry):
        rows0[i, pl.ds(0, 16)] = z16
        rows0[i, pl.ds(16, 16)] = z16
        return carry

    lax.fori_loop(0, 128, zf, 0)
    for k in range(24):
        pltpu.sync_copy(rows0, out_acc.at[pl.ds(base + k * 128, 128)])
    pltpu.sync_copy(rows0.at[pl.ds(0, 64)], out_acc.at[pl.ds(base + 3072, 64)])
    plsc.subcore_barrier()

    def body(jj, carry):
        j0 = wid * NCHUNK + jj * 2
        pltpu.sync_copy(src_hbm.at[pl.ds(j0, 2)], s_idx)
        pltpu.sync_copy(dst_hbm.at[pl.ds(j0, 2)], d_idx)
        da = pltpu.async_copy(y_hbm.at[s_idx.at[0]], rows0, sem0)
        db = pltpu.async_copy(y_hbm.at[s_idx.at[1]], rows1, sem1)
        da.wait()
        pltpu.sync_copy(rows0, out_acc.at[d_idx.at[0]], add=True)
        db.wait()
        pltpu.sync_copy(rows1, out_acc.at[d_idx.at[1]], add=True)
        return carry

    lax.fori_loop(0, NCHUNK // 2, body, 0)
    plsc.subcore_barrier()

    pltpu.sync_copy(out_acc.at[pl.ds(base, BLK)],
                    out_hbm.at[pl.ds(cid * N_PAD + base, BLK)])


def _make_sc_segsum():
    mesh = plsc.VectorSubcoreMesh(core_axis_name="c", subcore_axis_name="s",
                                  num_cores=2, num_subcores=16)
    return pl.kernel(
        _sc_body,
        out_type=jax.ShapeDtypeStruct((2 * N_PAD, 32), jnp.float32),
        mesh=mesh,
        compiler_params=pltpu.CompilerParams(use_tc_tiling_on_sc=False),
        scratch_types=[
            pltpu.VMEM((2, ECH), jnp.int32),
            pltpu.VMEM((2, ECH), jnp.int32),
            pltpu.VMEM((ECH, 32), jnp.float32),
            pltpu.VMEM((ECH, 32), jnp.float32),
            pltpu.VMEM_SHARED((N_PAD, 32), jnp.float32),
            pltpu.SemaphoreType.DMA,
            pltpu.SemaphoreType.DMA,
        ],
    )


_sc_segsum_fn = functools.cache(_make_sc_segsum)


def _bn_body(r_ref, s1_ref, s2_ref, o_ref):
    m = s1_ref[...] * (1.0 / N)
    v = s2_ref[...] * (1.0 / N) - m * m
    inv = lax.rsqrt(v + EPS)
    o_ref[...] = (r_ref[...] - m) * inv


def _gin_body(h_ref, p0_ref, p1_ref, w1_ref, b1_ref, w2_ref, b2_ref,
              r_ref, s1_ref, s2_ref):
    i = pl.program_id(0)
    z = h_ref[...] + p0_ref[...] + p1_ref[...]
    z1 = jnp.maximum(jnp.dot(z, w1_ref[...],
                             preferred_element_type=jnp.float32)
                     + b1_ref[...], 0.0)
    z2 = jnp.dot(z1, w2_ref[...], preferred_element_type=jnp.float32)
    r = jnp.maximum(z2 + b2_ref[...], 0.0)
    r_ref[...] = r
    row = lax.broadcasted_iota(jnp.int32, (BLK, 1), 0) + i * BLK
    rm = jnp.where(row < N, r, 0.0)

    @pl.when(i == 0)
    def _():
        s1_ref[...] = jnp.zeros_like(s1_ref)
        s2_ref[...] = jnp.zeros_like(s2_ref)

    s1_ref[...] += jnp.sum(rm, axis=0, keepdims=True)
    s2_ref[...] += jnp.sum(rm * rm, axis=0, keepdims=True)


def _pool_body(r_ref, s1_ref, s2_ref, bt_ref, o_ref):
    i = pl.program_id(0)
    m = s1_ref[...] * (1.0 / N)
    v = s2_ref[...] * (1.0 / N) - m * m
    inv = lax.rsqrt(v + EPS)
    hn = (r_ref[...] - m) * inv
    oh = (bt_ref[...] == lax.broadcasted_iota(jnp.int32, (BLK, B), 1))
    oh = oh.astype(jnp.float32)

    @pl.when(i == 0)
    def _():
        o_ref[...] = jnp.zeros_like(o_ref)

    o_ref[...] += lax.dot_general(oh, hn, (((0,), (0,)), ((), ())),
                                  preferred_element_type=jnp.float32,
                                  precision=HI)


def _gtens_body(t_ref, w_ref, g_ref):
    for j in range(8):
        row = t_ref[j, :]
        oh = (row[:, None] == lax.broadcasted_iota(jnp.int32, (1000, 32), 1))
        oh = oh.astype(jnp.float32)
        g_ref[j] = lax.dot_general(oh, w_ref[...], (((0,), (0,)), ((), ())),
                                   preferred_element_type=jnp.float32,
                                   precision=HI)


def _conv_body(g_ref, tb_ref, cb_ref, o_ref):
    accs = []
    for j in range(8):
        acc = jnp.zeros((32, 121), jnp.float32)
        for t in range(8):
            gt = g_ref[j, :, t * 32:(t + 1) * 32]
            tbt = tb_ref[:, t:t + 121]
            acc += lax.dot_general(gt, tbt, (((0,), (0,)), ((), ())),
                                   preferred_element_type=jnp.float32,
                                   precision=HI)
        accs.append(acc + cb_ref[...])
    o_ref[...] = jnp.concatenate(accs, axis=0)


def _head_body(pd_ref, cf_ref, fw_ref, fb_ref, wxd_ref, bxd_ref,
               c1a_ref, c1b_ref, b1_ref,
               c2_ref, b2_ref, c3_ref, b3_ref, o_ref):
    xd = jnp.maximum(jnp.dot(pd_ref[...], wxd_ref[...],
                             preferred_element_type=jnp.float32)
                     + bxd_ref[...], 0.0)
    xt = jnp.dot(cf_ref[...], fw_ref[...],
                 preferred_element_type=jnp.float32) + fb_ref[...]
    o1 = jnp.maximum(
        jnp.dot(xd, c1a_ref[...], preferred_element_type=jnp.float32)
        + jnp.dot(xt, c1b_ref[...], preferred_element_type=jnp.float32)
        + b1_ref[...], 0.0)
    o2 = jnp.maximum(jnp.dot(o1, c2_ref[...],
                             preferred_element_type=jnp.float32)
                     + b2_ref[...], 0.0)
    o_ref[...] = jnp.dot(o2, c3_ref[...],
                         preferred_element_type=jnp.float32) + b3_ref[...]


def _blk(shape, imap):
    return pl.BlockSpec(shape, imap)


_FULL = lambda shape: pl.BlockSpec(shape, lambda i: (0,) * len(shape))


def _bn_call(r, s1, s2):
    return pl.pallas_call(
        _bn_body,
        grid=(NB,),
        in_specs=[_blk((BLK, 32), lambda i: (i, 0)), _FULL((1, 32)),
                  _FULL((1, 32))],
        out_specs=_blk((BLK, 32), lambda i: (i, 0)),
        out_shape=jax.ShapeDtypeStruct((N_PAD, 32), jnp.float32),
    )(r, s1, s2)


def _gin_call(h, p0, p1, w1, b1, w2, b2):
    d = h.shape[1]
    return pl.pallas_call(
        _gin_body,
        grid=(NB,),
        in_specs=[_blk((BLK, d), lambda i: (i, 0)),
                  _blk((BLK, d), lambda i: (i, 0)),
                  _blk((BLK, d), lambda i: (i, 0)),
                  _FULL((d, 32)), _FULL((1, 32)),
                  _FULL((32, 32)), _FULL((1, 32))],
        out_specs=[_blk((BLK, 32), lambda i: (i, 0)),
                   _FULL((1, 32)), _FULL((1, 32))],
        out_shape=[jax.ShapeDtypeStruct((N_PAD, 32), jnp.float32),
                   jax.ShapeDtypeStruct((1, 32), jnp.float32),
                   jax.ShapeDtypeStruct((1, 32), jnp.float32)],
    )(h, p0, p1, w1, b1, w2, b2)


def _pool_call(r, s1, s2, bt):
    return pl.pallas_call(
        _pool_body,
        grid=(NB,),
        in_specs=[_blk((BLK, 32), lambda i: (i, 0)), _FULL((1, 32)),
                  _FULL((1, 32)), _blk((BLK, 1), lambda i: (i, 0))],
        out_specs=_FULL((B, 32)),
        out_shape=jax.ShapeDtypeStruct((B, 32), jnp.float32),
    )(r, s1, s2, bt)


def _gtens_call(target, wr):
    return pl.pallas_call(
        _gtens_body,
        grid=(NB,),
        in_specs=[_blk((8, 1000), lambda i: (i, 0)), _FULL((1000, 256))],
        out_specs=_blk((8, 32, 256), lambda i: (i, 0, 0)),
        out_shape=jax.ShapeDtypeStruct((B, 32, 256), jnp.float32),
    )(target, wr)


def _conv_call(g, tb, cb):
    return pl.pallas_call(
        _conv_body,
        grid=(NB,),
        in_specs=[_blk((8, 32, 256), lambda i: (i, 0, 0)), _FULL((32, 128)),
                  _FULL((32, 1))],
        out_specs=_blk((256, 121), lambda i: (i, 0)),
        out_shape=jax.ShapeDtypeStruct((B * 32, 121), jnp.float32),
    )(g, tb, cb)


def _head_call(pooled, cf, p):
    args = (pooled, cf,
            p['fc1_xt_W'], p['fc1_xt_b'].reshape(1, 128),
            p['fc1_xd_W'], p['fc1_xd_b'].reshape(1, 128),
            p['c1_W'][:128], p['c1_W'][128:], p['c1_b'].reshape(1, 1024),
            p['c2_W'], p['c2_b'].reshape(1, 256),
            p['c3_W'], p['c3_b'].reshape(1, 1))
    return pl.pallas_call(
        _head_body,
        out_shape=jax.ShapeDtypeStruct((B, 1), jnp.float32),
    )(*args)


def kernel(x, params, edge_index, batch, target):
    p = params
    src = edge_index[0].astype(jnp.int32)
    dst = edge_index[1].astype(jnp.int32)
    src = jnp.concatenate([src, jnp.zeros((E_PAD - E,), jnp.int32)])
    dst = jnp.concatenate([dst, jnp.full((E_PAD - E,), N, jnp.int32)])
    srcr = src.reshape(NW * NCHUNK, ECH)
    dstr = dst.reshape(NW * NCHUNK, ECH)

    segsum = _sc_segsum_fn()

    x_pad = jnp.zeros((N_PAD, 64), jnp.float32).at[:N, :55].set(x)
    w1, b1, w2, b2 = p['gin0']
    w1_pad = jnp.zeros((64, 32), jnp.float32).at[:55].set(w1)
    pa = segsum(x_pad[:, :32], srcr, dstr)
    pb = segsum(x_pad[:, 32:], srcr, dstr)
    p0 = jnp.concatenate([pa[:N_PAD], pb[:N_PAD]], axis=1)
    p1 = jnp.concatenate([pa[N_PAD:], pb[N_PAD:]], axis=1)
    r, s1, s2 = _gin_call(x_pad, p0, p1, w1_pad, b1.reshape(1, 32),
                          w2, b2.reshape(1, 32))

    for li in range(1, 5):
        w1, b1, w2, b2 = p['gin%d' % li]
        hn = _bn_call(r, s1, s2)
        pp = segsum(hn, srcr, dstr)
        r, s1, s2 = _gin_call(hn, pp[:N_PAD], pp[N_PAD:], w1,
                              b1.reshape(1, 32), w2, b2.reshape(1, 32))

    bt = jnp.full((N_PAD, 1), B, jnp.int32).at[:N, 0].set(batch.astype(jnp.int32))
    pooled = _pool_call(r, s1, s2, bt)

    wr = p['conv_xt_W'].transpose(1, 2, 0).reshape(1000, 256)
    tb = jnp.zeros((32, 128), jnp.float32).at[:26].set(p['emb_xt'])
    g = _gtens_call(target.astype(jnp.int32), wr)
    conv = _conv_call(g, tb, p['conv_xt_b'].reshape(32, 1))
    cf = conv.reshape(B, 32 * 121)

    return _head_call(pooled, cf, p)
